# baseline (speedup 1.0000x reference)
import functools

import jax
import jax.numpy as jnp
from jax import lax
from jax.experimental import pallas as pl
from jax.experimental.pallas import tpu as pltpu
from jax.experimental.pallas import tpu_sc as plsc

NUM_CORES = 2
NUM_SUBCORES = 16
NUM_WORKERS = NUM_CORES * NUM_SUBCORES

CH_A = 1024
NCH_A = 31
CH_B = 2048


def _mesh():
    return plsc.VectorSubcoreMesh(
        core_axis_name="c", subcore_axis_name="s", num_cores=NUM_CORES
    )


TC_W = 2048


def _transpose_table(tab_t, *, v, d):
    rows_per_blk = TC_W * d // 128

    def tr(in_ref, out_ref):
        x = in_ref[...].reshape(d, TC_W // 8, 8)
        out_ref[...] = jnp.transpose(x, (1, 2, 0)).reshape(rows_per_blk, 128)

    return pl.pallas_call(
        tr,
        grid=(pl.cdiv(v, TC_W),),
        in_specs=[pl.BlockSpec((d, TC_W), lambda c: (0, c))],
        out_specs=pl.BlockSpec((rows_per_blk, 128), lambda c: (c, 0)),
        out_shape=jax.ShapeDtypeStruct((v * d // 128, 128), jnp.float32),
    )(tab_t)


def _gather(idx_flat, tab_lin, *, n, d, bdim):
    per_w = n // NUM_WORKERS
    steps = per_w // CH_B
    fdim = n // bdim

    @functools.partial(
        pl.kernel,
        mesh=_mesh(),
        out_type=jax.ShapeDtypeStruct((fdim, bdim, d), jnp.float32),
        compiler_params=pltpu.CompilerParams(use_tc_tiling_on_sc=False),
        scratch_types=[
            pltpu.VMEM((2, CH_B), jnp.int32),
            pltpu.VMEM((2, CH_B, d), jnp.float32),
            pltpu.SemaphoreType.DMA((2,)),
            pltpu.SemaphoreType.DMA((2,)),
            pltpu.SemaphoreType.DMA((2,)),
        ],
    )
    def emb(idx_hbm, tab_hbm, out_hbm, idx_v, rows_v, sem_idx, sem_gat, sem_out):
        wid = lax.axis_index("s") * NUM_CORES + lax.axis_index("c")
        r0 = wid * per_w

        def load_idx(i, s):
            return pltpu.async_copy(
                idx_hbm.at[pl.ds(r0 + i * CH_B, CH_B)], idx_v.at[s], sem_idx.at[s]
            )

        idx_cp = [None, None]
        out_cp = [None, None]
        idx_cp[0] = load_idx(0, 0)
        for i in range(steps):
            s = i % 2
            if i + 1 < steps:
                idx_cp[1 - s] = load_idx(i + 1, 1 - s)
            idx_cp[s].wait()
            if out_cp[s] is not None:
                out_cp[s].wait()
            pltpu.async_copy(
                tab_hbm.at[idx_v.at[s]], rows_v.at[s], sem_gat.at[s]
            ).wait()
            r = r0 + i * CH_B
            ff = r // bdim
            b0 = r - ff * bdim
            out_cp[s] = pltpu.async_copy(
                rows_v.at[s], out_hbm.at[ff, pl.ds(b0, CH_B)], sem_out.at[s]
            )
        for cp in out_cp:
            if cp is not None:
                cp.wait()

    return emb(idx_flat, tab_lin)


@functools.partial(jax.jit, static_argnames=("b", "f", "d", "v"))
def _emb_lookup(idx_flat, tab_t, *, b, f, d, v):
    tab_lin = _transpose_table(tab_t, v=v, d=d).reshape(v, d)
    out_fbd = _gather(idx_flat, tab_lin, n=b * f, d=d, bdim=b)
    return out_fbd.transpose(1, 0, 2)


def kernel(indices, table):
    b, f = indices.shape
    v, d = table.shape
    idx_flat = indices.T.reshape(f * b)
    return _emb_lookup(idx_flat, table.T, b=b, f=f, d=d, v=v)


# speedup vs baseline: 1.1183x; 1.1183x over previous
import functools

import jax
import jax.numpy as jnp
from jax import lax
from jax.experimental import pallas as pl
from jax.experimental.pallas import tpu as pltpu
from jax.experimental.pallas import tpu_sc as plsc

NUM_CORES = 2
NUM_SUBCORES = 16
NUM_WORKERS = NUM_CORES * NUM_SUBCORES

CH_A = 1024
NCH_A = 31
CH_B = 2048


def _mesh():
    return plsc.VectorSubcoreMesh(
        core_axis_name="c", subcore_axis_name="s", num_cores=NUM_CORES
    )


TC_W = 2048


def _transpose_table(tab_t, *, v, d):
    def tr(in_ref, out_ref):
        out_ref[...] = in_ref[...].T

    return pl.pallas_call(
        tr,
        grid=(pl.cdiv(v, TC_W),),
        in_specs=[pl.BlockSpec((d, TC_W), lambda c: (0, c))],
        out_specs=pl.BlockSpec((TC_W, d), lambda c: (c, 0)),
        out_shape=jax.ShapeDtypeStruct((v, d), jnp.float32),
    )(tab_t)


def _gather(idx_flat, tab_lin, *, n, d, bdim):
    per_w = n // NUM_WORKERS
    steps = per_w // CH_B
    fdim = n // bdim

    @functools.partial(
        pl.kernel,
        mesh=_mesh(),
        out_type=jax.ShapeDtypeStruct((fdim, bdim, d), jnp.float32),
        compiler_params=pltpu.CompilerParams(use_tc_tiling_on_sc=False),
        scratch_types=[
            pltpu.VMEM((2, CH_B), jnp.int32),
            pltpu.VMEM((2, CH_B, d), jnp.float32),
            pltpu.SemaphoreType.DMA((2,)),
            pltpu.SemaphoreType.DMA((2,)),
            pltpu.SemaphoreType.DMA((2,)),
        ],
    )
    def emb(idx_hbm, tab_hbm, out_hbm, idx_v, rows_v, sem_idx, sem_gat, sem_out):
        wid = lax.axis_index("s") * NUM_CORES + lax.axis_index("c")
        r0 = wid * per_w

        def load_idx(i, s):
            return pltpu.async_copy(
                idx_hbm.at[pl.ds(r0 + i * CH_B, CH_B)], idx_v.at[s], sem_idx.at[s]
            )

        idx_cp = [None, None]
        out_cp = [None, None]
        idx_cp[0] = load_idx(0, 0)
        for i in range(steps):
            s = i % 2
            if i + 1 < steps:
                idx_cp[1 - s] = load_idx(i + 1, 1 - s)
            idx_cp[s].wait()
            if out_cp[s] is not None:
                out_cp[s].wait()
            pltpu.async_copy(
                tab_hbm.at[idx_v.at[s]], rows_v.at[s], sem_gat.at[s]
            ).wait()
            r = r0 + i * CH_B
            ff = r // bdim
            b0 = r - ff * bdim
            out_cp[s] = pltpu.async_copy(
                rows_v.at[s], out_hbm.at[ff, pl.ds(b0, CH_B)], sem_out.at[s]
            )
        for cp in out_cp:
            if cp is not None:
                cp.wait()

    return emb(idx_flat, tab_lin)


@functools.partial(jax.jit, static_argnames=("b", "f", "d", "v"))
def _emb_lookup(idx_flat, tab_t, *, b, f, d, v):
    tab_lin = _transpose_table(tab_t, v=v, d=d)
    out_fbd = _gather(idx_flat, tab_lin, n=b * f, d=d, bdim=b)
    return out_fbd.transpose(1, 0, 2)


def kernel(indices, table):
    b, f = indices.shape
    v, d = table.shape
    idx_flat = indices.T.reshape(f * b)
    return _emb_lookup(idx_flat, table.T, b=b, f=f, d=d, v=v)


# speedup vs baseline: 1.1288x; 1.0093x over previous
import functools

import jax
import jax.numpy as jnp
from jax import lax
from jax.experimental import pallas as pl
from jax.experimental.pallas import tpu as pltpu
from jax.experimental.pallas import tpu_sc as plsc

NUM_CORES = 2
NUM_SUBCORES = 16
NUM_WORKERS = NUM_CORES * NUM_SUBCORES
BATCH_PER_CHUNK = 32


@functools.partial(jax.jit, static_argnames=("b", "f", "d"))
def _emb_lookup(idx_flat, table, *, b, f, d):
    batch_per_w = b // NUM_WORKERS
    steps = batch_per_w // BATCH_PER_CHUNK
    chunk = BATCH_PER_CHUNK * f
    mesh = plsc.VectorSubcoreMesh(
        core_axis_name="c", subcore_axis_name="s", num_cores=NUM_CORES
    )

    @functools.partial(
        pl.kernel,
        mesh=mesh,
        out_type=jax.ShapeDtypeStruct((b, f, d), jnp.float32),
        compiler_params=pltpu.CompilerParams(use_tc_tiling_on_sc=False),
        scratch_types=[
            pltpu.VMEM((2, chunk), jnp.int32),
            pltpu.VMEM((2, chunk, d), jnp.float32),
            pltpu.SemaphoreType.DMA((2,)),
            pltpu.SemaphoreType.DMA((2,)),
            pltpu.SemaphoreType.DMA((2,)),
        ],
    )
    def emb(idx_hbm, tab_hbm, out_hbm, idx_v, rows_v, sem_idx, sem_gat, sem_out):
        wid = lax.axis_index("s") * NUM_CORES + lax.axis_index("c")
        b0 = wid * batch_per_w

        def load_idx(i, s):
            return pltpu.async_copy(
                idx_hbm.at[pl.ds((b0 + i * BATCH_PER_CHUNK) * f, chunk)],
                idx_v.at[s],
                sem_idx.at[s],
            )

        idx_cp = [None, None]
        out_cps = [[], []]
        idx_cp[0] = load_idx(0, 0)
        for i in range(steps):
            s = i % 2
            if i + 1 < steps:
                idx_cp[1 - s] = load_idx(i + 1, 1 - s)
            idx_cp[s].wait()
            for cp in out_cps[s]:
                cp.wait()
            out_cps[s] = []
            pltpu.async_copy(
                tab_hbm.at[idx_v.at[s]], rows_v.at[s], sem_gat.at[s]
            ).wait()
            for k in range(BATCH_PER_CHUNK):
                out_cps[s].append(
                    pltpu.async_copy(
                        rows_v.at[s].at[pl.ds(k * f, f)],
                        out_hbm.at[b0 + i * BATCH_PER_CHUNK + k],
                        sem_out.at[s],
                    )
                )
        for cps in out_cps:
            for cp in cps:
                cp.wait()

    return emb(idx_flat, table)


def kernel(indices, table):
    b, f = indices.shape
    v, d = table.shape
    return _emb_lookup(indices.reshape(b * f), table, b=b, f=f, d=d)
